# Initial kernel scaffold; baseline (speedup 1.0000x reference)
import jax
import jax.numpy as jnp
from jax.experimental import pallas as pl


def kernel(feature, attack_feature, W1, b1, W2, b2, edge_index):
    raise NotImplementedError("write your pallas kernel here")


# speedup vs baseline: 5.5265x; 5.5265x over previous
import dataclasses
import functools

import jax
import jax.numpy as jnp
from jax import lax
from jax.experimental import pallas as pl
from jax.experimental.pallas import tpu as pltpu
from jax.experimental.pallas import tpu_sc as plsc

N = 10000
E = 320000
IN_DIM = 128
HID_DIM = 128
OUT_DIM = 64
NODE = 500

NC = 2
NS = 16
NW = NC * NS
L = 16
C = 128
HR = 80
NP = HR * C

_mesh = plsc.VectorSubcoreMesh(core_axis_name="c", subcore_axis_name="s")

_cp = pltpu.CompilerParams()
if "needs_layout_passes" in pltpu.CompilerParams.__dataclass_fields__:
    _cp = dataclasses.replace(_cp, needs_layout_passes=False)


def _deg_body(src_hbm, dst_hbm, outs_hbm, outd_hbm,
              idx_v, hs_v, hd_v, id_v, accs_sh, accd_sh):
    cid = lax.axis_index("c")
    sid = lax.axis_index("s")
    wid = cid * NS + sid
    zvec = jnp.zeros((L,), jnp.float32)
    ones = jnp.ones((L,), jnp.float32)

    @pl.loop(0, HR)
    def _(r):
        for cc in range(C // L):
            hs_v[r, pl.ds(cc * L, L)] = zvec
            hd_v[r, pl.ds(cc * L, L)] = zvec

    for k in range(HR // L):
        id_v[pl.ds(k * L, L)] = lax.iota(jnp.int32, L) + k * L

    @pl.loop(sid * 8, HR, step=NS * 8)
    def _(r0):
        pltpu.sync_copy(hs_v.at[pl.ds(0, 8)], accs_sh.at[pl.ds(r0, 8)])
        pltpu.sync_copy(hd_v.at[pl.ds(0, 8)], accd_sh.at[pl.ds(r0, 8)])

    @pl.loop(wid * C, E, step=NW * C)
    def _(base):
        pltpu.sync_copy(src_hbm.at[pl.ds(base, C)], idx_v)
        for j in range(C // L):
            v = idx_v[pl.ds(j * L, L)]
            plsc.addupdate_scatter(hs_v, [v >> 7, v & 127], ones)
        pltpu.sync_copy(dst_hbm.at[pl.ds(base, C)], idx_v)
        for j in range(C // L):
            v = idx_v[pl.ds(j * L, L)]
            plsc.addupdate_scatter(hd_v, [v >> 7, v & 127], ones)

    plsc.subcore_barrier()
    pltpu.sync_copy(hs_v, accs_sh.at[id_v], add=True)
    pltpu.sync_copy(hd_v, accd_sh.at[id_v], add=True)
    plsc.subcore_barrier()
    @pl.loop(sid * 8, HR, step=NS * 8)
    def _(r0):
        pltpu.sync_copy(accs_sh.at[pl.ds(r0, 8)], hs_v.at[pl.ds(0, 8)])
        pltpu.sync_copy(hs_v.at[pl.ds(0, 8)], outs_hbm.at[cid, pl.ds(r0, 8)])
        pltpu.sync_copy(accd_sh.at[pl.ds(r0, 8)], hd_v.at[pl.ds(0, 8)])
        pltpu.sync_copy(hd_v.at[pl.ds(0, 8)], outd_hbm.at[cid, pl.ds(r0, 8)])


_deg = pl.kernel(
    _deg_body,
    out_type=[jax.ShapeDtypeStruct((NC, HR, C), jnp.float32),
              jax.ShapeDtypeStruct((NC, HR, C), jnp.float32)],
    mesh=_mesh,
    scratch_types=[
        pltpu.VMEM((C,), jnp.int32),
        pltpu.VMEM((HR, C), jnp.float32),
        pltpu.VMEM((HR, C), jnp.float32),
        pltpu.VMEM((HR,), jnp.int32),
        pltpu.VMEM_SHARED((HR, C), jnp.float32),
        pltpu.VMEM_SHARED((HR, C), jnp.float32),
    ],
    compiler_params=_cp,
)


def _make_spmm(d):
    def body(h_hbm, src_hbm, dst_hbm, out_hbm, srcv, dstv, rows, acc_sh, sem):
        cid = lax.axis_index("c")
        sid = lax.axis_index("s")
        wid = cid * NS + sid
        zvec = jnp.zeros((L,), jnp.float32)

        @pl.loop(0, C)
        def _(r):
            for cc in range(d // L):
                rows[r, pl.ds(cc * L, L)] = zvec

        @pl.loop(sid * C, NP, step=NS * C)
        def _(r0):
            pltpu.sync_copy(rows, acc_sh.at[pl.ds(r0, C)])

        plsc.subcore_barrier()

        @pl.loop(wid * C, E, step=NW * C)
        def _(base):
            pltpu.sync_copy(src_hbm.at[pl.ds(base, C)], srcv)
            pltpu.sync_copy(dst_hbm.at[pl.ds(base, C)], dstv)
            pltpu.async_copy(h_hbm.at[srcv], rows, sem).wait()
            pltpu.sync_copy(rows, acc_sh.at[dstv], add=True)

        plsc.subcore_barrier()

        @pl.loop(sid * C, NP, step=NS * C)
        def _(r0):
            pltpu.sync_copy(acc_sh.at[pl.ds(r0, C)], rows)
            pltpu.sync_copy(rows, out_hbm.at[cid, pl.ds(r0, C)])

    return pl.kernel(
        body,
        out_type=jax.ShapeDtypeStruct((NC, NP, d), jnp.float32),
        mesh=_mesh,
        scratch_types=[
            pltpu.VMEM((C,), jnp.int32),
            pltpu.VMEM((C,), jnp.int32),
            pltpu.VMEM((C, d), jnp.float32),
            pltpu.VMEM_SHARED((NP, d), jnp.float32),
            pltpu.SemaphoreType.DMA,
        ],
    )


_spmm_h = _make_spmm(HID_DIM)


_RB = 2000


def _mm_body(x_ref, w_ref, o_ref):
    o_ref[...] = jnp.dot(x_ref[...], w_ref[...],
                         preferred_element_type=jnp.float32)


_mm = pl.pallas_call(
    _mm_body,
    out_shape=jax.ShapeDtypeStruct((N, HID_DIM), jnp.float32),
    grid=(N // _RB,),
    in_specs=[pl.BlockSpec((_RB, IN_DIM), lambda i: (i, 0)),
              pl.BlockSpec((IN_DIM, HID_DIM), lambda i: (0, 0))],
    out_specs=pl.BlockSpec((_RB, HID_DIM), lambda i: (i, 0)),
)


def _norm_body(ds_ref, dd_ref, ns_ref, nd_ref):
    ns_ref[...] = lax.rsqrt(jnp.maximum(ds_ref[0] + ds_ref[1], 1.0))
    nd_ref[...] = lax.rsqrt(jnp.maximum(dd_ref[0] + dd_ref[1], 1.0))


_norm = pl.pallas_call(
    _norm_body,
    out_shape=[jax.ShapeDtypeStruct((HR, C), jnp.float32),
               jax.ShapeDtypeStruct((HR, C), jnp.float32)],
)


def _scale_body(a_ref, s_ref, o_ref):
    o_ref[...] = a_ref[...] * s_ref[...]


_scale = pl.pallas_call(
    _scale_body,
    out_shape=jax.ShapeDtypeStruct((N, HID_DIM), jnp.float32),
    grid=(N // _RB,),
    in_specs=[pl.BlockSpec((_RB, HID_DIM), lambda i: (i, 0)),
              pl.BlockSpec((_RB, 1), lambda i: (i, 0))],
    out_specs=pl.BlockSpec((_RB, HID_DIM), lambda i: (i, 0)),
)


def _mid_body(a_ref, nd_ref, ns_ref, b1_ref, o_ref):
    t = jnp.maximum((a_ref[0] + a_ref[1]) * nd_ref[...] + b1_ref[...], 0.0)
    o_ref[...] = t * ns_ref[...]


_mid = pl.pallas_call(
    _mid_body,
    out_shape=jax.ShapeDtypeStruct((N, HID_DIM), jnp.float32),
    grid=(N // _RB,),
    in_specs=[pl.BlockSpec((NC, _RB, HID_DIM), lambda i: (0, i, 0)),
              pl.BlockSpec((_RB, 1), lambda i: (i, 0)),
              pl.BlockSpec((_RB, 1), lambda i: (i, 0)),
              pl.BlockSpec((1, HID_DIM), lambda i: (0, 0))],
    out_specs=pl.BlockSpec((_RB, HID_DIM), lambda i: (i, 0)),
)


def _out_body(a_ref, w2_ref, nd_ref, b2_ref, o_ref):
    m = jnp.dot(a_ref[0] + a_ref[1], w2_ref[...],
                preferred_element_type=jnp.float32)
    o_ref[...] = m * nd_ref[...] + b2_ref[...]


_fin = pl.pallas_call(
    _out_body,
    out_shape=jax.ShapeDtypeStruct((N, OUT_DIM), jnp.float32),
    grid=(N // _RB,),
    in_specs=[pl.BlockSpec((NC, _RB, HID_DIM), lambda i: (0, i, 0)),
              pl.BlockSpec((HID_DIM, OUT_DIM), lambda i: (0, 0)),
              pl.BlockSpec((_RB, 1), lambda i: (i, 0)),
              pl.BlockSpec((1, OUT_DIM), lambda i: (0, 0))],
    out_specs=pl.BlockSpec((_RB, OUT_DIM), lambda i: (i, 0)),
)


def kernel(feature, attack_feature, W1, b1, W2, b2, edge_index):
    src = edge_index[0]
    dst = edge_index[1]
    x = jnp.concatenate([feature[:-NODE], attack_feature], axis=0)

    ds_p, dd_p = _deg(src, dst)
    xw1 = _mm(x, W1)
    ns8, nd8 = _norm(ds_p, dd_p)
    ns = ns8.reshape(-1)[:N, None]
    nd = nd8.reshape(-1)[:N, None]
    h = _scale(xw1, ns)
    agg1 = _spmm_h(h, src, dst)
    t = _mid(agg1, nd, ns, b1.reshape(1, -1))
    agg2 = _spmm_h(t, src, dst)
    return _fin(agg2, W2, nd, b2.reshape(1, -1))


# speedup vs baseline: 9.4305x; 1.7064x over previous
import dataclasses
import functools

import jax
import jax.numpy as jnp
from jax import lax
from jax.experimental import pallas as pl
from jax.experimental.pallas import tpu as pltpu
from jax.experimental.pallas import tpu_sc as plsc

N = 10000
E = 320000
IN_DIM = 128
HID_DIM = 128
OUT_DIM = 64
NODE = 500

NC = 2
NS = 16
NW = NC * NS
L = 16
C = 128
HR = 80
NP = HR * C
CPW = 80
EP = NW * CPW * C
NBUF = 2

_mesh = plsc.VectorSubcoreMesh(core_axis_name="c", subcore_axis_name="s")

_cp = pltpu.CompilerParams()
if "needs_layout_passes" in pltpu.CompilerParams.__dataclass_fields__:
    _cp = dataclasses.replace(_cp, needs_layout_passes=False)


def _deg_body(src3, dst3, outs_hbm, outd_hbm,
              idxs_v, idxd_v, hs_v, hd_v, id_v, accs_sh, accd_sh, sem):
    cid = lax.axis_index("c")
    sid = lax.axis_index("s")
    wid = cid * NS + sid
    zvec = jnp.zeros((L,), jnp.float32)
    ones = jnp.ones((L,), jnp.float32)

    i1 = pltpu.async_copy(src3.at[wid], idxs_v, sem)
    i2 = pltpu.async_copy(dst3.at[wid], idxd_v, sem)

    @pl.loop(0, HR)
    def _(r):
        for cc in range(C // L):
            hs_v[r, pl.ds(cc * L, L)] = zvec
            hd_v[r, pl.ds(cc * L, L)] = zvec

    for k in range(HR // L):
        id_v[pl.ds(k * L, L)] = lax.iota(jnp.int32, L) + k * L

    @pl.loop(sid * 8, HR, step=NS * 8)
    def _(r0):
        pltpu.sync_copy(hs_v.at[pl.ds(0, 8)], accs_sh.at[pl.ds(r0, 8)])
        pltpu.sync_copy(hd_v.at[pl.ds(0, 8)], accd_sh.at[pl.ds(r0, 8)])

    i1.wait()
    i2.wait()

    @pl.loop(0, CPW)
    def _(j):
        for g in range(C // L):
            v = idxs_v[j, pl.ds(g * L, L)]
            plsc.addupdate_scatter(hs_v, [v >> 7, v & 127], ones)
            w = idxd_v[j, pl.ds(g * L, L)]
            plsc.addupdate_scatter(hd_v, [w >> 7, w & 127], ones)

    plsc.subcore_barrier()
    pltpu.sync_copy(hs_v, accs_sh.at[id_v], add=True)
    pltpu.sync_copy(hd_v, accd_sh.at[id_v], add=True)
    plsc.subcore_barrier()
    @pl.loop(sid * 8, HR, step=NS * 8)
    def _(r0):
        pltpu.sync_copy(accs_sh.at[pl.ds(r0, 8)], hs_v.at[pl.ds(0, 8)])
        pltpu.sync_copy(hs_v.at[pl.ds(0, 8)], outs_hbm.at[cid, pl.ds(r0, 8)])
        pltpu.sync_copy(accd_sh.at[pl.ds(r0, 8)], hd_v.at[pl.ds(0, 8)])
        pltpu.sync_copy(hd_v.at[pl.ds(0, 8)], outd_hbm.at[cid, pl.ds(r0, 8)])


_deg = pl.kernel(
    _deg_body,
    out_type=[jax.ShapeDtypeStruct((NC, HR, C), jnp.float32),
              jax.ShapeDtypeStruct((NC, HR, C), jnp.float32)],
    mesh=_mesh,
    scratch_types=[
        pltpu.VMEM((CPW, C), jnp.int32),
        pltpu.VMEM((CPW, C), jnp.int32),
        pltpu.VMEM((HR, C), jnp.float32),
        pltpu.VMEM((HR, C), jnp.float32),
        pltpu.VMEM((HR,), jnp.int32),
        pltpu.VMEM_SHARED((HR, C), jnp.float32),
        pltpu.VMEM_SHARED((HR, C), jnp.float32),
        pltpu.SemaphoreType.DMA,
    ],
    compiler_params=_cp,
)


def _spmm_body(h_hbm, srcp, dstp, out_hbm, *rest):
    sbuf = rest[0:4]
    dbuf = rest[4:8]
    rows = rest[8:10]
    acc_sh = rest[10]
    isems = rest[11:15]
    gsems = rest[15:17]
    ssems = rest[17:19]
    cid = lax.axis_index("c")
    sid = lax.axis_index("s")
    wid = cid * NS + sid
    ebase = wid * (CPW * C)
    zvec = jnp.zeros((L,), jnp.float32)

    def idx_issue(c, k):
        pltpu.async_copy(srcp.at[pl.ds(ebase + c * C, C)], sbuf[k], isems[k])
        pltpu.async_copy(dstp.at[pl.ds(ebase + c * C, C)], dbuf[k], isems[k])

    def idx_drain(c, k):
        pltpu.make_async_copy(
            srcp.at[pl.ds(ebase + c * C, C)], sbuf[k], isems[k]).wait()
        pltpu.make_async_copy(
            dstp.at[pl.ds(ebase + c * C, C)], dbuf[k], isems[k]).wait()

    def gather(k, b):
        pltpu.async_copy(h_hbm.at[sbuf[k]], rows[b], gsems[b])

    def gather_drain(k, b):
        pltpu.make_async_copy(h_hbm.at[sbuf[k]], rows[b], gsems[b]).wait()

    def scatter(k, b):
        return pltpu.async_copy(rows[b], acc_sh.at[dbuf[k]], ssems[b],
                                add=True)

    for k in range(4):
        idx_issue(k, k)

    @pl.loop(0, C)
    def _(r):
        for cc in range(HID_DIM // L):
            rows[0][r, pl.ds(cc * L, L)] = zvec

    @pl.loop(sid * C, NP, step=NS * C)
    def _(r0):
        pltpu.sync_copy(rows[0], acc_sh.at[pl.ds(r0, C)])

    plsc.subcore_barrier()

    @pl.loop(0, CPW - 4, step=4)
    def _(j):
        idx_drain(j + 0, 0)
        gather(0, 0)
        idx_drain(j + 1, 1)
        gather(1, 1)
        gather_drain(0, 0)
        s0 = scatter(0, 0)
        gather_drain(1, 1)
        s1 = scatter(1, 1)
        s0.wait()
        idx_issue(j + 4, 0)
        idx_drain(j + 2, 2)
        gather(2, 0)
        s1.wait()
        idx_issue(j + 5, 1)
        idx_drain(j + 3, 3)
        gather(3, 1)
        gather_drain(2, 0)
        s2 = scatter(2, 0)
        gather_drain(3, 1)
        s3 = scatter(3, 1)
        s2.wait()
        idx_issue(j + 6, 2)
        s3.wait()
        idx_issue(j + 7, 3)

    jl = CPW - 4
    idx_drain(jl + 0, 0)
    gather(0, 0)
    idx_drain(jl + 1, 1)
    gather(1, 1)
    gather_drain(0, 0)
    s0 = scatter(0, 0)
    gather_drain(1, 1)
    s1 = scatter(1, 1)
    s0.wait()
    idx_drain(jl + 2, 2)
    gather(2, 0)
    s1.wait()
    idx_drain(jl + 3, 3)
    gather(3, 1)
    gather_drain(2, 0)
    s2 = scatter(2, 0)
    gather_drain(3, 1)
    s3 = scatter(3, 1)
    s2.wait()
    s3.wait()

    plsc.subcore_barrier()

    @pl.loop(sid * C, NP, step=NS * C)
    def _(r0):
        pltpu.sync_copy(acc_sh.at[pl.ds(r0, C)], rows[0])
        pltpu.sync_copy(rows[0], out_hbm.at[cid, pl.ds(r0, C)])


_spmm = pl.kernel(
    _spmm_body,
    out_type=jax.ShapeDtypeStruct((NC, NP, HID_DIM), jnp.float32),
    mesh=_mesh,
    scratch_types=(
        [pltpu.VMEM((C,), jnp.int32) for _ in range(8)]
        + [pltpu.VMEM((C, HID_DIM), jnp.float32) for _ in range(2)]
        + [pltpu.VMEM_SHARED((NP, HID_DIM), jnp.float32)]
        + [pltpu.SemaphoreType.DMA for _ in range(8)]
    ),
    compiler_params=_cp,
)


_RB = 2048


def _mm_body(x_ref, w_ref, o_ref):
    o_ref[...] = jnp.dot(x_ref[...], w_ref[...],
                         preferred_element_type=jnp.float32)


_mm = pl.pallas_call(
    _mm_body,
    out_shape=jax.ShapeDtypeStruct((NP, HID_DIM), jnp.float32),
    grid=(NP // _RB,),
    in_specs=[pl.BlockSpec((_RB, IN_DIM), lambda i: (i, 0)),
              pl.BlockSpec((IN_DIM, HID_DIM), lambda i: (0, 0))],
    out_specs=pl.BlockSpec((_RB, HID_DIM), lambda i: (i, 0)),
)


def _norm_body(ds_ref, dd_ref, ns_ref, nd_ref):
    ns_ref[...] = lax.rsqrt(jnp.maximum(ds_ref[0] + ds_ref[1], 1.0))
    nd_ref[...] = lax.rsqrt(jnp.maximum(dd_ref[0] + dd_ref[1], 1.0))


_norm = pl.pallas_call(
    _norm_body,
    out_shape=[jax.ShapeDtypeStruct((HR, C), jnp.float32),
               jax.ShapeDtypeStruct((HR, C), jnp.float32)],
)


def _scale_body(a_ref, s_ref, o_ref):
    o_ref[...] = a_ref[...] * s_ref[...]


_scale = pl.pallas_call(
    _scale_body,
    out_shape=jax.ShapeDtypeStruct((NP, HID_DIM), jnp.float32),
    grid=(NP // _RB,),
    in_specs=[pl.BlockSpec((_RB, HID_DIM), lambda i: (i, 0)),
              pl.BlockSpec((_RB, 1), lambda i: (i, 0))],
    out_specs=pl.BlockSpec((_RB, HID_DIM), lambda i: (i, 0)),
)


def _mid_body(a_ref, nd_ref, ns_ref, b1_ref, o_ref):
    t = jnp.maximum((a_ref[0] + a_ref[1]) * nd_ref[...] + b1_ref[...], 0.0)
    o_ref[...] = t * ns_ref[...]


_mid = pl.pallas_call(
    _mid_body,
    out_shape=jax.ShapeDtypeStruct((NP, HID_DIM), jnp.float32),
    grid=(NP // _RB,),
    in_specs=[pl.BlockSpec((NC, _RB, HID_DIM), lambda i: (0, i, 0)),
              pl.BlockSpec((_RB, 1), lambda i: (i, 0)),
              pl.BlockSpec((_RB, 1), lambda i: (i, 0)),
              pl.BlockSpec((1, HID_DIM), lambda i: (0, 0))],
    out_specs=pl.BlockSpec((_RB, HID_DIM), lambda i: (i, 0)),
)


_RBF = 2000


def _out_body(a_ref, w2_ref, nd_ref, b2_ref, o_ref):
    m = jnp.dot(a_ref[0] + a_ref[1], w2_ref[...],
                preferred_element_type=jnp.float32)
    o_ref[...] = m * nd_ref[...] + b2_ref[...]


_fin = pl.pallas_call(
    _out_body,
    out_shape=jax.ShapeDtypeStruct((N, OUT_DIM), jnp.float32),
    grid=(N // _RBF,),
    in_specs=[pl.BlockSpec((NC, _RBF, HID_DIM), lambda i: (0, i, 0)),
              pl.BlockSpec((HID_DIM, OUT_DIM), lambda i: (0, 0)),
              pl.BlockSpec((_RBF, 1), lambda i: (i, 0)),
              pl.BlockSpec((1, OUT_DIM), lambda i: (0, 0))],
    out_specs=pl.BlockSpec((_RBF, OUT_DIM), lambda i: (i, 0)),
)


def kernel(feature, attack_feature, W1, b1, W2, b2, edge_index):
    src = edge_index[0]
    dst = edge_index[1]
    pad_ids = N + (jnp.arange(EP - E, dtype=jnp.int32) % (NP - N))
    srcp = jnp.concatenate([src, pad_ids])
    dstp = jnp.concatenate([dst, pad_ids])
    src3 = srcp.reshape(NW, CPW, C)
    dst3 = dstp.reshape(NW, CPW, C)
    x = jnp.concatenate(
        [feature[:-NODE], attack_feature,
         jnp.zeros((NP - N, IN_DIM), feature.dtype)], axis=0)

    ds_p, dd_p = _deg(src3, dst3)
    xw1 = _mm(x, W1)
    ns8, nd8 = _norm(ds_p, dd_p)
    ns = ns8.reshape(-1)[:, None]
    nd = nd8.reshape(-1)[:, None]
    h = _scale(xw1, ns)
    agg1 = _spmm(h, srcp, dstp)
    t = _mid(agg1, nd, ns, b1.reshape(1, -1))
    agg2 = _spmm(t, srcp, dstp)
    return _fin(agg2, W2, nd[:N], b2.reshape(1, -1))


# speedup vs baseline: 9.5680x; 1.0146x over previous
import dataclasses
import functools

import jax
import jax.numpy as jnp
from jax import lax
from jax.experimental import pallas as pl
from jax.experimental.pallas import tpu as pltpu
from jax.experimental.pallas import tpu_sc as plsc

N = 10000
E = 320000
IN_DIM = 128
HID_DIM = 128
OUT_DIM = 64
NODE = 500

NC = 2
NS = 16
NW = NC * NS
L = 16
C = 128
HR = 80
NP = HR * C
CPW = 80
EP = NW * CPW * C
CS = 80
CPS = 128
NRB = 4

_mesh = plsc.VectorSubcoreMesh(core_axis_name="c", subcore_axis_name="s")

_cp = pltpu.CompilerParams()
if "needs_layout_passes" in pltpu.CompilerParams.__dataclass_fields__:
    _cp = dataclasses.replace(_cp, needs_layout_passes=False)


def _deg_body(src3, dst3, outs_hbm, outd_hbm,
              idxs_v, idxd_v, hs_v, hd_v, id_v, accs_sh, accd_sh, sem):
    cid = lax.axis_index("c")
    sid = lax.axis_index("s")
    wid = cid * NS + sid
    zvec = jnp.zeros((L,), jnp.float32)
    ones = jnp.ones((L,), jnp.float32)

    i1 = pltpu.async_copy(src3.at[wid], idxs_v, sem)
    i2 = pltpu.async_copy(dst3.at[wid], idxd_v, sem)

    @pl.loop(0, HR)
    def _(r):
        for cc in range(C // L):
            hs_v[r, pl.ds(cc * L, L)] = zvec
            hd_v[r, pl.ds(cc * L, L)] = zvec

    for k in range(HR // L):
        id_v[pl.ds(k * L, L)] = lax.iota(jnp.int32, L) + k * L

    @pl.loop(sid * 8, HR, step=NS * 8)
    def _(r0):
        pltpu.sync_copy(hs_v.at[pl.ds(0, 8)], accs_sh.at[pl.ds(r0, 8)])
        pltpu.sync_copy(hd_v.at[pl.ds(0, 8)], accd_sh.at[pl.ds(r0, 8)])

    i1.wait()
    i2.wait()

    @pl.loop(0, CPW)
    def _(j):
        for g in range(C // L):
            v = idxs_v[j, pl.ds(g * L, L)]
            plsc.addupdate_scatter(hs_v, [v >> 7, v & 127], ones)
            w = idxd_v[j, pl.ds(g * L, L)]
            plsc.addupdate_scatter(hd_v, [w >> 7, w & 127], ones)

    plsc.subcore_barrier()
    pltpu.sync_copy(hs_v, accs_sh.at[id_v], add=True)
    pltpu.sync_copy(hd_v, accd_sh.at[id_v], add=True)
    plsc.subcore_barrier()
    @pl.loop(sid * 8, HR, step=NS * 8)
    def _(r0):
        pltpu.sync_copy(accs_sh.at[pl.ds(r0, 8)], hs_v.at[pl.ds(0, 8)])
        pltpu.sync_copy(hs_v.at[pl.ds(0, 8)], outs_hbm.at[cid, pl.ds(r0, 8)])
        pltpu.sync_copy(accd_sh.at[pl.ds(r0, 8)], hd_v.at[pl.ds(0, 8)])
        pltpu.sync_copy(hd_v.at[pl.ds(0, 8)], outd_hbm.at[cid, pl.ds(r0, 8)])


_deg = pl.kernel(
    _deg_body,
    out_type=[jax.ShapeDtypeStruct((NC, HR, C), jnp.float32),
              jax.ShapeDtypeStruct((NC, HR, C), jnp.float32)],
    mesh=_mesh,
    scratch_types=[
        pltpu.VMEM((CPW, C), jnp.int32),
        pltpu.VMEM((CPW, C), jnp.int32),
        pltpu.VMEM((HR, C), jnp.float32),
        pltpu.VMEM((HR, C), jnp.float32),
        pltpu.VMEM((HR,), jnp.int32),
        pltpu.VMEM_SHARED((HR, C), jnp.float32),
        pltpu.VMEM_SHARED((HR, C), jnp.float32),
        pltpu.SemaphoreType.DMA,
    ],
    compiler_params=_cp,
)


def _spmm_body(h_hbm, srcp, dstp, out_hbm, *rest):
    sbuf = rest[0:8]
    dbuf = rest[8:16]
    rows = rest[16:16 + NRB]
    acc_sh = rest[16 + NRB]
    isems = rest[17 + NRB:25 + NRB]
    gsems = rest[25 + NRB:25 + 2 * NRB]
    ssems = rest[25 + 2 * NRB:25 + 3 * NRB]
    cid = lax.axis_index("c")
    sid = lax.axis_index("s")
    wid = cid * NS + sid
    ebase = wid * (CPS * CS)
    zvec = jnp.zeros((L,), jnp.float32)

    def idx_issue(c, k):
        pltpu.async_copy(srcp.at[pl.ds(ebase + c * CS, CS)], sbuf[k], isems[k])
        pltpu.async_copy(dstp.at[pl.ds(ebase + c * CS, CS)], dbuf[k], isems[k])

    def idx_drain(c, k):
        pltpu.make_async_copy(
            srcp.at[pl.ds(ebase + c * CS, CS)], sbuf[k], isems[k]).wait()
        pltpu.make_async_copy(
            dstp.at[pl.ds(ebase + c * CS, CS)], dbuf[k], isems[k]).wait()

    def gather(k, b):
        pltpu.async_copy(h_hbm.at[sbuf[k]], rows[b], gsems[b])

    def gather_drain(k, b):
        pltpu.make_async_copy(h_hbm.at[sbuf[k]], rows[b], gsems[b]).wait()

    def scatter(k, b):
        return pltpu.async_copy(rows[b], acc_sh.at[dbuf[k]], ssems[b],
                                add=True)

    for k in range(8):
        idx_issue(k, k)

    @pl.loop(0, CS)
    def _(r):
        for cc in range(HID_DIM // L):
            rows[0][r, pl.ds(cc * L, L)] = zvec

    @pl.loop(sid * CS, NP, step=NS * CS)
    def _(r0):
        pltpu.sync_copy(rows[0], acc_sh.at[pl.ds(r0, CS)])

    plsc.subcore_barrier()

    def _group(j, half, prefetch):
        for k in range(NRB):
            idx_drain(j + half * NRB + k, half * NRB + k)
            gather(half * NRB + k, k)
        ss = []
        for k in range(NRB):
            gather_drain(half * NRB + k, k)
            ss.append(scatter(half * NRB + k, k))
        for k in range(NRB):
            ss[k].wait()
            if prefetch:
                idx_issue(j + 8 + half * NRB + k, half * NRB + k)

    @pl.loop(0, CPS - 8, step=8)
    def _(j):
        _group(j, 0, True)
        _group(j, 1, True)

    _group(CPS - 8, 0, False)
    _group(CPS - 8, 1, False)

    plsc.subcore_barrier()

    @pl.loop(sid * CS, NP, step=NS * CS)
    def _(r0):
        pltpu.sync_copy(acc_sh.at[pl.ds(r0, CS)], rows[0])
        pltpu.sync_copy(rows[0], out_hbm.at[cid, pl.ds(r0, CS)])


_spmm = pl.kernel(
    _spmm_body,
    out_type=jax.ShapeDtypeStruct((NC, NP, HID_DIM), jnp.float32),
    mesh=_mesh,
    scratch_types=(
        [pltpu.VMEM((CS,), jnp.int32) for _ in range(16)]
        + [pltpu.VMEM((CS, HID_DIM), jnp.float32) for _ in range(NRB)]
        + [pltpu.VMEM_SHARED((NP, HID_DIM), jnp.float32)]
        + [pltpu.SemaphoreType.DMA for _ in range(8 + 2 * NRB)]
    ),
    compiler_params=_cp,
)


_RB = 2048


def _mm_body(x_ref, w_ref, o_ref):
    o_ref[...] = jnp.dot(x_ref[...], w_ref[...],
                         preferred_element_type=jnp.float32)


_mm = pl.pallas_call(
    _mm_body,
    out_shape=jax.ShapeDtypeStruct((NP, HID_DIM), jnp.float32),
    grid=(NP // _RB,),
    in_specs=[pl.BlockSpec((_RB, IN_DIM), lambda i: (i, 0)),
              pl.BlockSpec((IN_DIM, HID_DIM), lambda i: (0, 0))],
    out_specs=pl.BlockSpec((_RB, HID_DIM), lambda i: (i, 0)),
)


def _norm_body(ds_ref, dd_ref, ns_ref, nd_ref):
    ns_ref[...] = lax.rsqrt(jnp.maximum(ds_ref[0] + ds_ref[1], 1.0))
    nd_ref[...] = lax.rsqrt(jnp.maximum(dd_ref[0] + dd_ref[1], 1.0))


_norm = pl.pallas_call(
    _norm_body,
    out_shape=[jax.ShapeDtypeStruct((HR, C), jnp.float32),
               jax.ShapeDtypeStruct((HR, C), jnp.float32)],
)


def _scale_body(a_ref, s_ref, o_ref):
    o_ref[...] = a_ref[...] * s_ref[...]


_scale = pl.pallas_call(
    _scale_body,
    out_shape=jax.ShapeDtypeStruct((NP, HID_DIM), jnp.float32),
    grid=(NP // _RB,),
    in_specs=[pl.BlockSpec((_RB, HID_DIM), lambda i: (i, 0)),
              pl.BlockSpec((_RB, 1), lambda i: (i, 0))],
    out_specs=pl.BlockSpec((_RB, HID_DIM), lambda i: (i, 0)),
)


def _mid_body(a_ref, nd_ref, ns_ref, b1_ref, o_ref):
    t = jnp.maximum((a_ref[0] + a_ref[1]) * nd_ref[...] + b1_ref[...], 0.0)
    o_ref[...] = t * ns_ref[...]


_mid = pl.pallas_call(
    _mid_body,
    out_shape=jax.ShapeDtypeStruct((NP, HID_DIM), jnp.float32),
    grid=(NP // _RB,),
    in_specs=[pl.BlockSpec((NC, _RB, HID_DIM), lambda i: (0, i, 0)),
              pl.BlockSpec((_RB, 1), lambda i: (i, 0)),
              pl.BlockSpec((_RB, 1), lambda i: (i, 0)),
              pl.BlockSpec((1, HID_DIM), lambda i: (0, 0))],
    out_specs=pl.BlockSpec((_RB, HID_DIM), lambda i: (i, 0)),
)


_RBF = 2000


def _out_body(a_ref, w2_ref, nd_ref, b2_ref, o_ref):
    m = jnp.dot(a_ref[0] + a_ref[1], w2_ref[...],
                preferred_element_type=jnp.float32)
    o_ref[...] = m * nd_ref[...] + b2_ref[...]


_fin = pl.pallas_call(
    _out_body,
    out_shape=jax.ShapeDtypeStruct((N, OUT_DIM), jnp.float32),
    grid=(N // _RBF,),
    in_specs=[pl.BlockSpec((NC, _RBF, HID_DIM), lambda i: (0, i, 0)),
              pl.BlockSpec((HID_DIM, OUT_DIM), lambda i: (0, 0)),
              pl.BlockSpec((_RBF, 1), lambda i: (i, 0)),
              pl.BlockSpec((1, OUT_DIM), lambda i: (0, 0))],
    out_specs=pl.BlockSpec((_RBF, OUT_DIM), lambda i: (i, 0)),
)


def kernel(feature, attack_feature, W1, b1, W2, b2, edge_index):
    src = edge_index[0]
    dst = edge_index[1]
    pad_ids = N + (jnp.arange(EP - E, dtype=jnp.int32) % (NP - N))
    srcp = jnp.concatenate([src, pad_ids])
    dstp = jnp.concatenate([dst, pad_ids])
    src3 = srcp.reshape(NW, CPW, C)
    dst3 = dstp.reshape(NW, CPW, C)
    x = jnp.concatenate(
        [feature[:-NODE], attack_feature,
         jnp.zeros((NP - N, IN_DIM), feature.dtype)], axis=0)

    ds_p, dd_p = _deg(src3, dst3)
    xw1 = _mm(x, W1)
    ns8, nd8 = _norm(ds_p, dd_p)
    ns = ns8.reshape(-1)[:, None]
    nd = nd8.reshape(-1)[:, None]
    h = _scale(xw1, ns)
    agg1 = _spmm(h, srcp, dstp)
    t = _mid(agg1, nd, ns, b1.reshape(1, -1))
    agg2 = _spmm(t, srcp, dstp)
    return _fin(agg2, W2, nd[:N], b2.reshape(1, -1))


# speedup vs baseline: 9.7035x; 1.0142x over previous
import dataclasses

import jax
import jax.numpy as jnp
from jax import lax
from jax.experimental import pallas as pl
from jax.experimental.pallas import tpu as pltpu
from jax.experimental.pallas import tpu_sc as plsc

N = 10000
E = 320000
IN_DIM = 128
HID_DIM = 128
OUT_DIM = 64
NODE = 500

NC = 2
NS = 16
NW = NC * NS
L = 16
HR = 80
EW = E // NW
CS = 80
CPS = EW // CS
NRB = 4

_mesh = plsc.VectorSubcoreMesh(core_axis_name="c", subcore_axis_name="s")

_cp = pltpu.CompilerParams()
if "needs_layout_passes" in pltpu.CompilerParams.__dataclass_fields__:
    _cp = dataclasses.replace(_cp, needs_layout_passes=False)


def _deg_body(src_hbm, dst_hbm, outs_hbm, outd_hbm,
              idxs_v, idxd_v, hs_v, hd_v, id_v, accs_sh, accd_sh, sem):
    cid = lax.axis_index("c")
    sid = lax.axis_index("s")
    wid = cid * NS + sid
    zvec = jnp.zeros((L,), jnp.float32)
    ones = jnp.ones((L,), jnp.float32)

    i1 = pltpu.async_copy(src_hbm.at[pl.ds(wid * EW, EW)], idxs_v, sem)
    i2 = pltpu.async_copy(dst_hbm.at[pl.ds(wid * EW, EW)], idxd_v, sem)

    @pl.loop(0, HR)
    def _(r):
        for cc in range(128 // L):
            hs_v[r, pl.ds(cc * L, L)] = zvec
            hd_v[r, pl.ds(cc * L, L)] = zvec

    for k in range(HR // L):
        id_v[pl.ds(k * L, L)] = lax.iota(jnp.int32, L) + k * L

    @pl.loop(sid * 8, HR, step=NS * 8)
    def _(r0):
        pltpu.sync_copy(hs_v.at[pl.ds(0, 8)], accs_sh.at[pl.ds(r0, 8)])
        pltpu.sync_copy(hd_v.at[pl.ds(0, 8)], accd_sh.at[pl.ds(r0, 8)])

    i1.wait()
    i2.wait()

    @pl.loop(0, EW, step=L)
    def _(o):
        v = idxs_v[pl.ds(o, L)]
        plsc.addupdate_scatter(hs_v, [v >> 7, v & 127], ones)
        w = idxd_v[pl.ds(o, L)]
        plsc.addupdate_scatter(hd_v, [w >> 7, w & 127], ones)

    plsc.subcore_barrier()
    pltpu.sync_copy(hs_v, accs_sh.at[id_v], add=True)
    pltpu.sync_copy(hd_v, accd_sh.at[id_v], add=True)
    plsc.subcore_barrier()
    @pl.loop(sid * 8, HR, step=NS * 8)
    def _(r0):
        pltpu.sync_copy(accs_sh.at[pl.ds(r0, 8)], hs_v.at[pl.ds(0, 8)])
        pltpu.sync_copy(hs_v.at[pl.ds(0, 8)], outs_hbm.at[cid, pl.ds(r0, 8)])
        pltpu.sync_copy(accd_sh.at[pl.ds(r0, 8)], hd_v.at[pl.ds(0, 8)])
        pltpu.sync_copy(hd_v.at[pl.ds(0, 8)], outd_hbm.at[cid, pl.ds(r0, 8)])


_deg = pl.kernel(
    _deg_body,
    out_type=[jax.ShapeDtypeStruct((NC, HR, 128), jnp.float32),
              jax.ShapeDtypeStruct((NC, HR, 128), jnp.float32)],
    mesh=_mesh,
    scratch_types=[
        pltpu.VMEM((EW,), jnp.int32),
        pltpu.VMEM((EW,), jnp.int32),
        pltpu.VMEM((HR, 128), jnp.float32),
        pltpu.VMEM((HR, 128), jnp.float32),
        pltpu.VMEM((HR,), jnp.int32),
        pltpu.VMEM_SHARED((HR, 128), jnp.float32),
        pltpu.VMEM_SHARED((HR, 128), jnp.float32),
        pltpu.SemaphoreType.DMA,
    ],
    compiler_params=_cp,
)


def _spmm_body(h_hbm, src_hbm, dst_hbm, out_hbm, *rest):
    sbuf = rest[0:8]
    dbuf = rest[8:16]
    rows = rest[16:16 + NRB]
    acc_sh = rest[16 + NRB]
    isems = rest[17 + NRB:25 + NRB]
    gsems = rest[25 + NRB:25 + 2 * NRB]
    ssems = rest[25 + 2 * NRB:25 + 3 * NRB]
    cid = lax.axis_index("c")
    sid = lax.axis_index("s")
    wid = cid * NS + sid
    ebase = wid * EW
    zvec = jnp.zeros((L,), jnp.float32)

    def idx_issue(c, k):
        pltpu.async_copy(
            src_hbm.at[pl.ds(ebase + c * CS, CS)], sbuf[k], isems[k])
        pltpu.async_copy(
            dst_hbm.at[pl.ds(ebase + c * CS, CS)], dbuf[k], isems[k])

    def idx_drain(c, k):
        pltpu.make_async_copy(
            src_hbm.at[pl.ds(ebase + c * CS, CS)], sbuf[k],
            isems[k]).wait()
        pltpu.make_async_copy(
            dst_hbm.at[pl.ds(ebase + c * CS, CS)], dbuf[k],
            isems[k]).wait()

    def gather(k, b):
        pltpu.async_copy(h_hbm.at[sbuf[k]], rows[b], gsems[b])

    def gather_drain(k, b):
        pltpu.make_async_copy(h_hbm.at[sbuf[k]], rows[b], gsems[b]).wait()

    def scatter(k, b):
        return pltpu.async_copy(rows[b], acc_sh.at[dbuf[k]], ssems[b],
                                add=True)

    for k in range(8):
        idx_issue(k, k)

    @pl.loop(0, CS)
    def _(r):
        for cc in range(HID_DIM // L):
            rows[0][r, pl.ds(cc * L, L)] = zvec

    @pl.loop(sid * CS, N, step=NS * CS)
    def _(r0):
        pltpu.sync_copy(rows[0], acc_sh.at[pl.ds(r0, CS)])

    plsc.subcore_barrier()

    def _group(j, half, pre):
        for k in range(NRB):
            idx_drain(j + half * NRB + k, half * NRB + k)
            gather(half * NRB + k, k)
        ss = []
        for k in range(NRB):
            gather_drain(half * NRB + k, k)
            ss.append(scatter(half * NRB + k, k))
        for k in range(NRB):
            ss[k].wait()
            if k < pre:
                idx_issue(j + 8 + half * NRB + k, half * NRB + k)

    @pl.loop(0, CPS - 13, step=8)
    def _(j):
        _group(j, 0, NRB)
        _group(j, 1, NRB)

    _group(CPS - 13, 0, NRB)
    _group(CPS - 13, 1, 1)

    for k in range(5):
        b = k % NRB
        idx_drain(CPS - 5 + k, k)
        gather(k, b)
        gather_drain(k, b)
        scatter(k, b).wait()

    plsc.subcore_barrier()

    @pl.loop(sid * CS, N, step=NS * CS)
    def _(r0):
        pltpu.sync_copy(acc_sh.at[pl.ds(r0, CS)], rows[0])
        pltpu.sync_copy(rows[0], out_hbm.at[cid, pl.ds(r0, CS)])


_spmm = pl.kernel(
    _spmm_body,
    out_type=jax.ShapeDtypeStruct((NC, N, HID_DIM), jnp.float32),
    mesh=_mesh,
    scratch_types=(
        [pltpu.VMEM((CS,), jnp.int32) for _ in range(16)]
        + [pltpu.VMEM((CS, HID_DIM), jnp.float32) for _ in range(NRB)]
        + [pltpu.VMEM_SHARED((N, HID_DIM), jnp.float32)]
        + [pltpu.SemaphoreType.DMA for _ in range(8 + 2 * NRB)]
    ),
    compiler_params=_cp,
)


_RB = 2000


def _mm_body(x_ref, w_ref, o_ref):
    o_ref[...] = jnp.dot(x_ref[...], w_ref[...],
                         preferred_element_type=jnp.float32)


_mm = pl.pallas_call(
    _mm_body,
    out_shape=jax.ShapeDtypeStruct((N, HID_DIM), jnp.float32),
    grid=(N // _RB,),
    in_specs=[pl.BlockSpec((_RB, IN_DIM), lambda i: (i, 0)),
              pl.BlockSpec((IN_DIM, HID_DIM), lambda i: (0, 0))],
    out_specs=pl.BlockSpec((_RB, HID_DIM), lambda i: (i, 0)),
)


def _norm_body(ds_ref, dd_ref, ns_ref, nd_ref):
    ns_ref[...] = lax.rsqrt(jnp.maximum(ds_ref[0] + ds_ref[1], 1.0))
    nd_ref[...] = lax.rsqrt(jnp.maximum(dd_ref[0] + dd_ref[1], 1.0))


_norm = pl.pallas_call(
    _norm_body,
    out_shape=[jax.ShapeDtypeStruct((HR, 128), jnp.float32),
               jax.ShapeDtypeStruct((HR, 128), jnp.float32)],
)


def _scale_body(a_ref, s_ref, o_ref):
    o_ref[...] = a_ref[...] * s_ref[...]


_scale = pl.pallas_call(
    _scale_body,
    out_shape=jax.ShapeDtypeStruct((N, HID_DIM), jnp.float32),
    grid=(N // _RB,),
    in_specs=[pl.BlockSpec((_RB, HID_DIM), lambda i: (i, 0)),
              pl.BlockSpec((_RB, 1), lambda i: (i, 0))],
    out_specs=pl.BlockSpec((_RB, HID_DIM), lambda i: (i, 0)),
)


def _mid_body(a_ref, nd_ref, ns_ref, b1_ref, o_ref):
    t = jnp.maximum((a_ref[0] + a_ref[1]) * nd_ref[...] + b1_ref[...], 0.0)
    o_ref[...] = t * ns_ref[...]


_mid = pl.pallas_call(
    _mid_body,
    out_shape=jax.ShapeDtypeStruct((N, HID_DIM), jnp.float32),
    grid=(N // _RB,),
    in_specs=[pl.BlockSpec((NC, _RB, HID_DIM), lambda i: (0, i, 0)),
              pl.BlockSpec((_RB, 1), lambda i: (i, 0)),
              pl.BlockSpec((_RB, 1), lambda i: (i, 0)),
              pl.BlockSpec((1, HID_DIM), lambda i: (0, 0))],
    out_specs=pl.BlockSpec((_RB, HID_DIM), lambda i: (i, 0)),
)


def _out_body(a_ref, w2_ref, nd_ref, b2_ref, o_ref):
    m = jnp.dot(a_ref[0] + a_ref[1], w2_ref[...],
                preferred_element_type=jnp.float32)
    o_ref[...] = m * nd_ref[...] + b2_ref[...]


_fin = pl.pallas_call(
    _out_body,
    out_shape=jax.ShapeDtypeStruct((N, OUT_DIM), jnp.float32),
    grid=(N // _RB,),
    in_specs=[pl.BlockSpec((NC, _RB, HID_DIM), lambda i: (0, i, 0)),
              pl.BlockSpec((HID_DIM, OUT_DIM), lambda i: (0, 0)),
              pl.BlockSpec((_RB, 1), lambda i: (i, 0)),
              pl.BlockSpec((1, OUT_DIM), lambda i: (0, 0))],
    out_specs=pl.BlockSpec((_RB, OUT_DIM), lambda i: (i, 0)),
)


def kernel(feature, attack_feature, W1, b1, W2, b2, edge_index):
    src = edge_index[0]
    dst = edge_index[1]
    x = jnp.concatenate([feature[:-NODE], attack_feature], axis=0)

    ds_p, dd_p = _deg(src, dst)
    xw1 = _mm(x, W1)
    ns8, nd8 = _norm(ds_p, dd_p)
    ns = ns8.reshape(-1)[:N, None]
    nd = nd8.reshape(-1)[:N, None]
    h = _scale(xw1, ns)
    agg1 = _spmm(h, src, dst)
    t = _mid(agg1, nd, ns, b1.reshape(1, -1))
    agg2 = _spmm(t, src, dst)
    return _fin(agg2, W2, nd, b2.reshape(1, -1))


# speedup vs baseline: 9.7124x; 1.0009x over previous
import dataclasses

import jax
import jax.numpy as jnp
from jax import lax
from jax.experimental import pallas as pl
from jax.experimental.pallas import tpu as pltpu
from jax.experimental.pallas import tpu_sc as plsc

N = 10000
E = 320000
IN_DIM = 128
HID_DIM = 128
OUT_DIM = 64
NODE = 500

NC = 2
NS = 16
NW = NC * NS
L = 16
HR = 80
EW = E // NW
CS = 80
CPS = EW // CS
NRB = 4

_mesh = plsc.VectorSubcoreMesh(core_axis_name="c", subcore_axis_name="s")

_cp = pltpu.CompilerParams()
if "needs_layout_passes" in pltpu.CompilerParams.__dataclass_fields__:
    _cp = dataclasses.replace(_cp, needs_layout_passes=False)


def _deg_body(src_hbm, dst_hbm, outs_hbm, outd_hbm,
              idxs_v, idxd_v, hs_v, hd_v, id_v, accs_sh, accd_sh, sem):
    cid = lax.axis_index("c")
    sid = lax.axis_index("s")
    wid = cid * NS + sid
    zvec = jnp.zeros((L,), jnp.float32)
    ones = jnp.ones((L,), jnp.float32)

    i1 = pltpu.async_copy(src_hbm.at[pl.ds(wid * EW, EW)], idxs_v, sem)
    i2 = pltpu.async_copy(dst_hbm.at[pl.ds(wid * EW, EW)], idxd_v, sem)

    @pl.loop(0, HR)
    def _(r):
        for cc in range(128 // L):
            hs_v[r, pl.ds(cc * L, L)] = zvec
            hd_v[r, pl.ds(cc * L, L)] = zvec

    for k in range(HR // L):
        id_v[pl.ds(k * L, L)] = lax.iota(jnp.int32, L) + k * L

    @pl.loop(sid * 8, HR, step=NS * 8)
    def _(r0):
        pltpu.sync_copy(hs_v.at[pl.ds(0, 8)], accs_sh.at[pl.ds(r0, 8)])
        pltpu.sync_copy(hd_v.at[pl.ds(0, 8)], accd_sh.at[pl.ds(r0, 8)])

    i1.wait()
    i2.wait()

    @pl.loop(0, EW, step=L)
    def _(o):
        v = idxs_v[pl.ds(o, L)]
        plsc.addupdate_scatter(hs_v, [v >> 7, v & 127], ones)
        w = idxd_v[pl.ds(o, L)]
        plsc.addupdate_scatter(hd_v, [w >> 7, w & 127], ones)

    plsc.subcore_barrier()
    pltpu.sync_copy(hs_v, accs_sh.at[id_v], add=True)
    pltpu.sync_copy(hd_v, accd_sh.at[id_v], add=True)
    plsc.subcore_barrier()
    @pl.loop(sid * 8, HR, step=NS * 8)
    def _(r0):
        pltpu.sync_copy(accs_sh.at[pl.ds(r0, 8)], hs_v.at[pl.ds(0, 8)])
        pltpu.sync_copy(hs_v.at[pl.ds(0, 8)], outs_hbm.at[cid, pl.ds(r0, 8)])
        pltpu.sync_copy(accd_sh.at[pl.ds(r0, 8)], hd_v.at[pl.ds(0, 8)])
        pltpu.sync_copy(hd_v.at[pl.ds(0, 8)], outd_hbm.at[cid, pl.ds(r0, 8)])


_deg = pl.kernel(
    _deg_body,
    out_type=[jax.ShapeDtypeStruct((NC, HR, 128), jnp.float32),
              jax.ShapeDtypeStruct((NC, HR, 128), jnp.float32)],
    mesh=_mesh,
    scratch_types=[
        pltpu.VMEM((EW,), jnp.int32),
        pltpu.VMEM((EW,), jnp.int32),
        pltpu.VMEM((HR, 128), jnp.float32),
        pltpu.VMEM((HR, 128), jnp.float32),
        pltpu.VMEM((HR,), jnp.int32),
        pltpu.VMEM_SHARED((HR, 128), jnp.float32),
        pltpu.VMEM_SHARED((HR, 128), jnp.float32),
        pltpu.SemaphoreType.DMA,
    ],
    compiler_params=_cp,
)


def _spmm_body(h_hbm, src_hbm, dst_hbm, out_hbm, *rest):
    sbuf = rest[0:8]
    dbuf = rest[8:16]
    rows = rest[16:16 + NRB]
    acc_sh = rest[16 + NRB]
    isems = rest[17 + NRB:25 + NRB]
    gsems = rest[25 + NRB:25 + 2 * NRB]
    ssems = rest[25 + 2 * NRB:25 + 3 * NRB]
    cid = lax.axis_index("c")
    sid = lax.axis_index("s")
    wid = cid * NS + sid
    ebase = wid * EW
    zvec = jnp.zeros((L,), jnp.float32)

    def idx_issue(c, k):
        pltpu.async_copy(
            src_hbm.at[pl.ds(ebase + c * CS, CS)], sbuf[k], isems[k])
        pltpu.async_copy(
            dst_hbm.at[pl.ds(ebase + c * CS, CS)], dbuf[k], isems[k])

    def idx_drain(c, k):
        pltpu.make_async_copy(
            src_hbm.at[pl.ds(ebase + c * CS, CS)], sbuf[k],
            isems[k]).wait()
        pltpu.make_async_copy(
            dst_hbm.at[pl.ds(ebase + c * CS, CS)], dbuf[k],
            isems[k]).wait()

    def gather(k, b):
        pltpu.async_copy(h_hbm.at[sbuf[k]], rows[b], gsems[b])

    def gather_drain(k, b):
        pltpu.make_async_copy(h_hbm.at[sbuf[k]], rows[b], gsems[b]).wait()

    def scatter(k, b):
        return pltpu.async_copy(rows[b], acc_sh.at[dbuf[k]], ssems[b],
                                add=True)

    for k in range(8):
        idx_issue(k, k)

    @pl.loop(0, CS)
    def _(r):
        for cc in range(HID_DIM // L):
            rows[0][r, pl.ds(cc * L, L)] = zvec

    @pl.loop(sid * CS, N, step=NS * CS)
    def _(r0):
        pltpu.sync_copy(rows[0], acc_sh.at[pl.ds(r0, CS)])

    plsc.subcore_barrier()

    def _group(j, half, pre):
        for k in range(NRB):
            idx_drain(j + half * NRB + k, half * NRB + k)
            gather(half * NRB + k, k)
        ss = []
        for k in range(NRB):
            gather_drain(half * NRB + k, k)
            ss.append(scatter(half * NRB + k, k))
        for k in range(NRB):
            ss[k].wait()
            if k < pre:
                idx_issue(j + 8 + half * NRB + k, half * NRB + k)

    @pl.loop(0, CPS - 13, step=8)
    def _(j):
        _group(j, 0, NRB)
        _group(j, 1, NRB)

    _group(CPS - 13, 0, NRB)
    _group(CPS - 13, 1, 1)

    for k in range(5):
        b = k % NRB
        idx_drain(CPS - 5 + k, k)
        gather(k, b)
        gather_drain(k, b)
        scatter(k, b).wait()

    plsc.subcore_barrier()

    @pl.loop(sid * CS, N, step=NS * CS)
    def _(r0):
        pltpu.sync_copy(acc_sh.at[pl.ds(r0, CS)], rows[0])
        pltpu.sync_copy(rows[0], out_hbm.at[cid, pl.ds(r0, CS)])


_spmm = pl.kernel(
    _spmm_body,
    out_type=jax.ShapeDtypeStruct((NC, N, HID_DIM), jnp.float32),
    mesh=_mesh,
    scratch_types=(
        [pltpu.VMEM((CS,), jnp.int32) for _ in range(16)]
        + [pltpu.VMEM((CS, HID_DIM), jnp.float32) for _ in range(NRB)]
        + [pltpu.VMEM_SHARED((N, HID_DIM), jnp.float32)]
        + [pltpu.SemaphoreType.DMA for _ in range(8 + 2 * NRB)]
    ),
    compiler_params=_cp,
)


_RB = 2000


def _mm_body(x_ref, w_ref, s_ref, o_ref):
    o_ref[...] = jnp.dot(x_ref[...], w_ref[...],
                         preferred_element_type=jnp.float32) * s_ref[...]


_mm = pl.pallas_call(
    _mm_body,
    out_shape=jax.ShapeDtypeStruct((N, HID_DIM), jnp.float32),
    grid=(N // _RB,),
    in_specs=[pl.BlockSpec((_RB, IN_DIM), lambda i: (i, 0)),
              pl.BlockSpec((IN_DIM, HID_DIM), lambda i: (0, 0)),
              pl.BlockSpec((_RB, 1), lambda i: (i, 0))],
    out_specs=pl.BlockSpec((_RB, HID_DIM), lambda i: (i, 0)),
)


def _norm_body(ds_ref, dd_ref, ns_ref, nd_ref):
    ns_ref[...] = lax.rsqrt(jnp.maximum(ds_ref[0] + ds_ref[1], 1.0))
    nd_ref[...] = lax.rsqrt(jnp.maximum(dd_ref[0] + dd_ref[1], 1.0))


_norm = pl.pallas_call(
    _norm_body,
    out_shape=[jax.ShapeDtypeStruct((HR, 128), jnp.float32),
               jax.ShapeDtypeStruct((HR, 128), jnp.float32)],
)


def _mid_body(a_ref, nd_ref, ns_ref, b1_ref, o_ref):
    t = jnp.maximum((a_ref[0] + a_ref[1]) * nd_ref[...] + b1_ref[...], 0.0)
    o_ref[...] = t * ns_ref[...]


_mid = pl.pallas_call(
    _mid_body,
    out_shape=jax.ShapeDtypeStruct((N, HID_DIM), jnp.float32),
    grid=(N // _RB,),
    in_specs=[pl.BlockSpec((NC, _RB, HID_DIM), lambda i: (0, i, 0)),
              pl.BlockSpec((_RB, 1), lambda i: (i, 0)),
              pl.BlockSpec((_RB, 1), lambda i: (i, 0)),
              pl.BlockSpec((1, HID_DIM), lambda i: (0, 0))],
    out_specs=pl.BlockSpec((_RB, HID_DIM), lambda i: (i, 0)),
)


def _out_body(a_ref, w2_ref, nd_ref, b2_ref, o_ref):
    m = jnp.dot(a_ref[0] + a_ref[1], w2_ref[...],
                preferred_element_type=jnp.float32)
    o_ref[...] = m * nd_ref[...] + b2_ref[...]


_fin = pl.pallas_call(
    _out_body,
    out_shape=jax.ShapeDtypeStruct((N, OUT_DIM), jnp.float32),
    grid=(N // _RB,),
    in_specs=[pl.BlockSpec((NC, _RB, HID_DIM), lambda i: (0, i, 0)),
              pl.BlockSpec((HID_DIM, OUT_DIM), lambda i: (0, 0)),
              pl.BlockSpec((_RB, 1), lambda i: (i, 0)),
              pl.BlockSpec((1, OUT_DIM), lambda i: (0, 0))],
    out_specs=pl.BlockSpec((_RB, OUT_DIM), lambda i: (i, 0)),
)


def kernel(feature, attack_feature, W1, b1, W2, b2, edge_index):
    src = edge_index[0]
    dst = edge_index[1]
    x = jnp.concatenate([feature[:-NODE], attack_feature], axis=0)

    ds_p, dd_p = _deg(src, dst)
    ns8, nd8 = _norm(ds_p, dd_p)
    ns = ns8.reshape(-1)[:N, None]
    nd = nd8.reshape(-1)[:N, None]
    h = _mm(x, W1, ns)
    agg1 = _spmm(h, src, dst)
    t = _mid(agg1, nd, ns, b1.reshape(1, -1))
    agg2 = _spmm(t, src, dst)
    return _fin(agg2, W2, nd, b2.reshape(1, -1))


# speedup vs baseline: 10.0605x; 1.0358x over previous
import dataclasses

import jax
import jax.numpy as jnp
from jax import lax
from jax.experimental import pallas as pl
from jax.experimental.pallas import tpu as pltpu
from jax.experimental.pallas import tpu_sc as plsc

N = 10000
E = 320000
IN_DIM = 128
HID_DIM = 128
OUT_DIM = 64
NODE = 500

NC = 2
NS = 16
NW = NC * NS
L = 16
HR = 80
EW = E // NW
CS = 80
CPS = EW // CS
NRB = 4

_mesh = plsc.VectorSubcoreMesh(core_axis_name="c", subcore_axis_name="s")

_cp = pltpu.CompilerParams()
if "needs_layout_passes" in pltpu.CompilerParams.__dataclass_fields__:
    _cp = dataclasses.replace(_cp, needs_layout_passes=False)


def _deg_body(src_hbm, dst_hbm, outs_hbm, outd_hbm,
              idxs_v, idxd_v, hs_v, hd_v, id_v, accs_sh, accd_sh, sem):
    cid = lax.axis_index("c")
    sid = lax.axis_index("s")
    wid = cid * NS + sid
    zvec = jnp.zeros((L,), jnp.float32)
    ones = jnp.ones((L,), jnp.float32)

    i1 = pltpu.async_copy(src_hbm.at[pl.ds(wid * EW, EW)], idxs_v, sem)
    i2 = pltpu.async_copy(dst_hbm.at[pl.ds(wid * EW, EW)], idxd_v, sem)

    @pl.loop(0, HR)
    def _(r):
        for cc in range(128 // L):
            hs_v[r, pl.ds(cc * L, L)] = zvec
            hd_v[r, pl.ds(cc * L, L)] = zvec

    for k in range(HR // L):
        id_v[pl.ds(k * L, L)] = lax.iota(jnp.int32, L) + k * L

    @pl.loop(sid * 8, HR, step=NS * 8)
    def _(r0):
        pltpu.sync_copy(hs_v.at[pl.ds(0, 8)], accs_sh.at[pl.ds(r0, 8)])
        pltpu.sync_copy(hd_v.at[pl.ds(0, 8)], accd_sh.at[pl.ds(r0, 8)])

    i1.wait()
    i2.wait()

    @pl.loop(0, EW, step=L)
    def _(o):
        v = idxs_v[pl.ds(o, L)]
        plsc.addupdate_scatter(hs_v, [v >> 7, v & 127], ones)
        w = idxd_v[pl.ds(o, L)]
        plsc.addupdate_scatter(hd_v, [w >> 7, w & 127], ones)

    plsc.subcore_barrier()
    pltpu.sync_copy(hs_v, accs_sh.at[id_v], add=True)
    pltpu.sync_copy(hd_v, accd_sh.at[id_v], add=True)
    plsc.subcore_barrier()
    @pl.loop(sid * 8, HR, step=NS * 8)
    def _(r0):
        pltpu.sync_copy(accs_sh.at[pl.ds(r0, 8)], hs_v.at[pl.ds(0, 8)])
        pltpu.sync_copy(hs_v.at[pl.ds(0, 8)], outs_hbm.at[cid, pl.ds(r0, 8)])
        pltpu.sync_copy(accd_sh.at[pl.ds(r0, 8)], hd_v.at[pl.ds(0, 8)])
        pltpu.sync_copy(hd_v.at[pl.ds(0, 8)], outd_hbm.at[cid, pl.ds(r0, 8)])


_deg = pl.kernel(
    _deg_body,
    out_type=[jax.ShapeDtypeStruct((NC, HR, 128), jnp.float32),
              jax.ShapeDtypeStruct((NC, HR, 128), jnp.float32)],
    mesh=_mesh,
    scratch_types=[
        pltpu.VMEM((EW,), jnp.int32),
        pltpu.VMEM((EW,), jnp.int32),
        pltpu.VMEM((HR, 128), jnp.float32),
        pltpu.VMEM((HR, 128), jnp.float32),
        pltpu.VMEM((HR,), jnp.int32),
        pltpu.VMEM_SHARED((HR, 128), jnp.float32),
        pltpu.VMEM_SHARED((HR, 128), jnp.float32),
        pltpu.SemaphoreType.DMA,
    ],
    compiler_params=_cp,
)


def _spmm_body(h_hbm, src_hbm, dst_hbm, out_hbm, *rest):
    sbuf = rest[0:8]
    dbuf = rest[8:16]
    rows = rest[16:16 + NRB]
    acc_sh = rest[16 + NRB]
    isems = rest[17 + NRB:25 + NRB]
    gsems = rest[25 + NRB:25 + 2 * NRB]
    ssems = rest[25 + 2 * NRB:25 + 3 * NRB]
    cid = lax.axis_index("c")
    sid = lax.axis_index("s")
    wid = cid * NS + sid
    ebase = wid * EW
    zvec = jnp.zeros((L,), jnp.float32)

    def idx_issue(c, k):
        pltpu.async_copy(
            src_hbm.at[pl.ds(ebase + c * CS, CS)], sbuf[k], isems[k])
        pltpu.async_copy(
            dst_hbm.at[pl.ds(ebase + c * CS, CS)], dbuf[k], isems[k])

    def idx_drain(c, k):
        pltpu.make_async_copy(
            src_hbm.at[pl.ds(ebase + c * CS, CS)], sbuf[k],
            isems[k]).wait()
        pltpu.make_async_copy(
            dst_hbm.at[pl.ds(ebase + c * CS, CS)], dbuf[k],
            isems[k]).wait()

    def gather(k, b):
        pltpu.async_copy(h_hbm.at[sbuf[k]], rows[b], gsems[b])

    def gather_drain(k, b):
        pltpu.make_async_copy(h_hbm.at[sbuf[k]], rows[b], gsems[b]).wait()

    def scatter(k, b):
        return pltpu.async_copy(rows[b], acc_sh.at[dbuf[k]], ssems[b],
                                add=True)

    for k in range(8):
        idx_issue(k, k)

    @pl.loop(0, CS)
    def _(r):
        for cc in range(HID_DIM // L):
            rows[0][r, pl.ds(cc * L, L)] = zvec

    @pl.loop(sid * CS, N, step=NS * CS)
    def _(r0):
        pltpu.sync_copy(rows[0], acc_sh.at[pl.ds(r0, CS)])

    plsc.subcore_barrier()

    def _group(j, half, pre):
        for k in range(NRB):
            idx_drain(j + half * NRB + k, half * NRB + k)
            gather(half * NRB + k, k)
        ss = []
        for k in range(NRB):
            gather_drain(half * NRB + k, k)
            ss.append(scatter(half * NRB + k, k))
        for k in range(NRB):
            ss[k].wait()
            if k < pre:
                idx_issue(j + 8 + half * NRB + k, half * NRB + k)

    @pl.loop(0, CPS - 13, step=8)
    def _(j):
        _group(j, 0, NRB)
        _group(j, 1, NRB)

    _group(CPS - 13, 0, NRB)
    _group(CPS - 13, 1, 1)

    for k in range(5):
        b = k % NRB
        idx_drain(CPS - 5 + k, k)
        gather(k, b)
        gather_drain(k, b)
        scatter(k, b).wait()

    plsc.subcore_barrier()

    @pl.loop(sid * CS, N, step=NS * CS)
    def _(r0):
        pltpu.sync_copy(acc_sh.at[pl.ds(r0, CS)], rows[0])
        pltpu.sync_copy(rows[0], out_hbm.at[cid, pl.ds(r0, CS)])


_spmm = pl.kernel(
    _spmm_body,
    out_type=jax.ShapeDtypeStruct((NC, N, HID_DIM), jnp.float32),
    mesh=_mesh,
    scratch_types=(
        [pltpu.VMEM((CS,), jnp.int32) for _ in range(16)]
        + [pltpu.VMEM((CS, HID_DIM), jnp.float32) for _ in range(NRB)]
        + [pltpu.VMEM_SHARED((N, HID_DIM), jnp.float32)]
        + [pltpu.SemaphoreType.DMA for _ in range(8 + 2 * NRB)]
    ),
    compiler_params=_cp,
)


_RB = 2000


def _mm_body(x_ref, w_ref, s_ref, o_ref):
    o_ref[...] = jnp.dot(x_ref[...], w_ref[...],
                         preferred_element_type=jnp.float32) * s_ref[...]


_mm = pl.pallas_call(
    _mm_body,
    out_shape=jax.ShapeDtypeStruct((N, HID_DIM), jnp.float32),
    grid=(N // _RB,),
    in_specs=[pl.BlockSpec((_RB, IN_DIM), lambda i: (i, 0)),
              pl.BlockSpec((IN_DIM, HID_DIM), lambda i: (0, 0)),
              pl.BlockSpec((_RB, 1), lambda i: (i, 0))],
    out_specs=pl.BlockSpec((_RB, HID_DIM), lambda i: (i, 0)),
)


def _norm_body(ds_ref, dd_ref, ns_ref, nd_ref):
    ns_ref[...] = lax.rsqrt(jnp.maximum(ds_ref[0] + ds_ref[1], 1.0))
    nd_ref[...] = lax.rsqrt(jnp.maximum(dd_ref[0] + dd_ref[1], 1.0))


_norm = pl.pallas_call(
    _norm_body,
    out_shape=[jax.ShapeDtypeStruct((HR, 128), jnp.float32),
               jax.ShapeDtypeStruct((HR, 128), jnp.float32)],
)


def _mid_body(a_ref, nd_ref, ns_ref, b1_ref, o_ref):
    t = jnp.maximum((a_ref[0] + a_ref[1]) * nd_ref[...] + b1_ref[...], 0.0)
    o_ref[...] = t * ns_ref[...]


_mid = pl.pallas_call(
    _mid_body,
    out_shape=jax.ShapeDtypeStruct((N, HID_DIM), jnp.float32),
    grid=(N // _RB,),
    in_specs=[pl.BlockSpec((NC, _RB, HID_DIM), lambda i: (0, i, 0)),
              pl.BlockSpec((_RB, 1), lambda i: (i, 0)),
              pl.BlockSpec((_RB, 1), lambda i: (i, 0)),
              pl.BlockSpec((1, HID_DIM), lambda i: (0, 0))],
    out_specs=pl.BlockSpec((_RB, HID_DIM), lambda i: (i, 0)),
)


def _out_body(a_ref, w2_ref, nd_ref, b2_ref, o_ref):
    m = jnp.dot(a_ref[0] + a_ref[1], w2_ref[...],
                preferred_element_type=jnp.float32)
    o_ref[...] = m * nd_ref[...] + b2_ref[...]


_fin = pl.pallas_call(
    _out_body,
    out_shape=jax.ShapeDtypeStruct((N, OUT_DIM), jnp.float32),
    grid=(N // _RB,),
    in_specs=[pl.BlockSpec((NC, _RB, HID_DIM), lambda i: (0, i, 0)),
              pl.BlockSpec((HID_DIM, OUT_DIM), lambda i: (0, 0)),
              pl.BlockSpec((_RB, 1), lambda i: (i, 0)),
              pl.BlockSpec((1, OUT_DIM), lambda i: (0, 0))],
    out_specs=pl.BlockSpec((_RB, OUT_DIM), lambda i: (i, 0)),
)


_EB = 64000


def _split_body(ab_ref, s_ref, d_ref):
    s_ref[...] = ab_ref[0]
    d_ref[...] = ab_ref[1]


_split = pl.pallas_call(
    _split_body,
    out_shape=[jax.ShapeDtypeStruct((E,), jnp.int32),
               jax.ShapeDtypeStruct((E,), jnp.int32)],
)


def kernel(feature, attack_feature, W1, b1, W2, b2, edge_index):
    src, dst = _split(edge_index)
    x = jnp.concatenate([feature[:-NODE], attack_feature], axis=0)

    ds_p, dd_p = _deg(src, dst)
    ns8, nd8 = _norm(ds_p, dd_p)
    ns = ns8.reshape(-1)[:N, None]
    nd = nd8.reshape(-1)[:N, None]
    h = _mm(x, W1, ns)
    agg1 = _spmm(h, src, dst)
    t = _mid(agg1, nd, ns, b1.reshape(1, -1))
    agg2 = _spmm(t, src, dst)
    return _fin(agg2, W2, nd, b2.reshape(1, -1))
